# Initial kernel scaffold; baseline (speedup 1.0000x reference)
import jax
import jax.numpy as jnp
from jax.experimental import pallas as pl


def kernel(feats, labels, centers_weight):
    raise NotImplementedError("write your pallas kernel here")


# speedup vs baseline: 1.7721x; 1.7721x over previous
import functools

import jax
import jax.numpy as jnp
from jax import lax
from jax.experimental import pallas as pl
from jax.experimental.pallas import tpu as pltpu
from jax.experimental.pallas import tpu_sc as plsc

_NUM_CLASSES = 1000
_D = 128
_B = 16384
_NC = 2
_NS = 16
_NW = _NC * _NS
_BPW = _B // _NW
_CHUNK = 256
_NCHUNK = _BPW // _CHUNK
_VPR = _D // 16


def _sc_body(feats_hbm, labels_hbm, centers_hbm, out_hbm,
             idx_v, feats_v, rows_v, acc_v, sem):
    wid = lax.axis_index("s") * _NC + lax.axis_index("c")
    base = wid * _BPW
    acc = tuple(jnp.zeros((16,), jnp.float32) for _ in range(_VPR))
    for c in range(_NCHUNK):
        rowbase = base + c * _CHUNK
        pltpu.sync_copy(labels_hbm.at[pl.ds(rowbase, _CHUNK)], idx_v)
        pltpu.async_copy(centers_hbm.at[idx_v], rows_v, sem).wait()
        pltpu.sync_copy(feats_hbm.at[pl.ds(rowbase, _CHUNK)], feats_v)

        def body(i, acc):
            out = []
            for j in range(_VPR):
                f = feats_v[i, pl.ds(j * 16, 16)]
                ctr = rows_v[i, pl.ds(j * 16, 16)]
                d = f - ctr
                out.append(acc[j] + d * d)
            return tuple(out)

        acc = lax.fori_loop(0, _CHUNK, body, acc)
    total = acc[0]
    for j in range(1, _VPR):
        total = total + acc[j]
    acc_v[...] = total
    pltpu.sync_copy(acc_v, out_hbm.at[wid])


@jax.jit
def kernel(feats, labels, centers_weight):
    labels = jnp.squeeze(labels).astype(jnp.int32)
    mesh = plsc.VectorSubcoreMesh(core_axis_name="c", subcore_axis_name="s")
    partial_fn = functools.partial(
        pl.kernel,
        mesh=mesh,
        out_type=jax.ShapeDtypeStruct((_NW, 16), jnp.float32),
        scratch_types=[
            pltpu.VMEM((_CHUNK,), jnp.int32),
            pltpu.VMEM((_CHUNK, _D), jnp.float32),
            pltpu.VMEM((_CHUNK, _D), jnp.float32),
            pltpu.VMEM((16,), jnp.float32),
            pltpu.SemaphoreType.DMA,
        ],
    )(_sc_body)
    partials = partial_fn(feats, labels, centers_weight)
    return jnp.sum(partials) / jnp.float32(_B * _D)


# speedup vs baseline: 1.9424x; 1.0961x over previous
import functools

import jax
import jax.numpy as jnp
from jax import lax
from jax.experimental import pallas as pl
from jax.experimental.pallas import tpu as pltpu
from jax.experimental.pallas import tpu_sc as plsc

_NUM_CLASSES = 1000
_D = 128
_B = 16384
_NC = 2
_NS = 16
_NW = _NC * _NS
_BPW = _B // _NW
_CH = 128
_NCHUNK = _BPW // _CH
_VPR = _D // 16
_UNROLL = 4


def _sc_body(feats_hbm, labels_hbm, centers_hbm, out_hbm,
             idx_v, rows0, rows1, feats0, feats1, acc_v,
             gsem0, gsem1, fsem0, fsem1):
    rows_v = (rows0, rows1)
    feats_v = (feats0, feats1)
    gsem = (gsem0, gsem1)
    fsem = (fsem0, fsem1)

    wid = lax.axis_index("s") * _NC + lax.axis_index("c")
    base = wid * _BPW
    pltpu.sync_copy(labels_hbm.at[pl.ds(wid * _NCHUNK, _NCHUNK)], idx_v)

    def start(c):
        buf = c % 2
        hg = pltpu.async_copy(centers_hbm.at[idx_v.at[c]], rows_v[buf], gsem[buf])
        hf = pltpu.async_copy(feats_hbm.at[pl.ds(base + c * _CH, _CH)],
                              feats_v[buf], fsem[buf])
        return hg, hf

    pending = [start(0), start(1)]
    acc = tuple(jnp.zeros((16,), jnp.float32) for _ in range(_VPR))
    for c in range(_NCHUNK):
        buf = c % 2
        hg, hf = pending[c]
        hg.wait()
        hf.wait()

        fv = feats_v[buf]
        rv = rows_v[buf]

        def body(it, acc, fv=fv, rv=rv):
            i = it * _UNROLL
            out = list(acc)
            for r in range(_UNROLL):
                for j in range(_VPR):
                    f = fv[i + r, pl.ds(j * 16, 16)]
                    ctr = rv[i + r, pl.ds(j * 16, 16)]
                    d = f - ctr
                    out[j] = out[j] + d * d
            return tuple(out)

        acc = lax.fori_loop(0, _CH // _UNROLL, body, acc)
        if c + 2 < _NCHUNK:
            pending.append(start(c + 2))

    total = acc[0]
    for j in range(1, _VPR):
        total = total + acc[j]
    acc_v[...] = total
    pltpu.sync_copy(acc_v, out_hbm.at[wid])


@jax.jit
def kernel(feats, labels, centers_weight):
    labels2d = jnp.squeeze(labels).astype(jnp.int32).reshape(_B // _CH, _CH)
    mesh = plsc.VectorSubcoreMesh(core_axis_name="c", subcore_axis_name="s")
    partial_fn = functools.partial(
        pl.kernel,
        mesh=mesh,
        out_type=jax.ShapeDtypeStruct((_NW, 16), jnp.float32),
        scratch_types=[
            pltpu.VMEM((_NCHUNK, _CH), jnp.int32),
            pltpu.VMEM((_CH, _D), jnp.float32),
            pltpu.VMEM((_CH, _D), jnp.float32),
            pltpu.VMEM((_CH, _D), jnp.float32),
            pltpu.VMEM((_CH, _D), jnp.float32),
            pltpu.VMEM((16,), jnp.float32),
            pltpu.SemaphoreType.DMA,
            pltpu.SemaphoreType.DMA,
            pltpu.SemaphoreType.DMA,
            pltpu.SemaphoreType.DMA,
        ],
    )(_sc_body)
    partials = partial_fn(feats, labels2d, centers_weight)
    return jnp.sum(partials) / jnp.float32(_B * _D)
